# initial kernel scaffold (unmeasured)
import jax
import jax.numpy as jnp
from jax import lax
from jax.experimental import pallas as pl
from jax.experimental.pallas import tpu as pltpu

N_DEV = 32
SQ = 256
HQ = 8
HKV = 2
GROUP = HQ // HKV
DH = 128
DM = HQ * DH
SCALE = 0.08838834764831843


def kernel(x, Wq, Wo, K_ext, V_ext):
    skv = K_ext.shape[1]

    def body(x_ref, wq_ref, wo_ref, k_ref, v_ref, out_ref,
             acc_o, acc_m, acc_l, obuf, mlbuf,
             send_o, recv_o, send_ml, recv_ml, credit_sem):
        my = lax.axis_index("i")
        left = lax.rem(my - 1 + N_DEV, N_DEV)
        right = lax.rem(my + 1, N_DEV)

        barrier = pltpu.get_barrier_semaphore()
        pl.semaphore_signal(barrier, inc=1, device_id=(left,),
                            device_id_type=pl.DeviceIdType.MESH)
        pl.semaphore_signal(barrier, inc=1, device_id=(right,),
                            device_id_type=pl.DeviceIdType.MESH)
        pl.semaphore_wait(barrier, 2)

        xv = x_ref[0].astype(jnp.bfloat16)
        wq = wq_ref[...].astype(jnp.bfloat16)
        q = jnp.dot(xv, wq, preferred_element_type=jnp.float32)

        kfull = k_ref[0]
        vfull = v_ref[0]
        for h in range(HQ):
            kvh = h // GROUP
            qh = (q[:, h * DH:(h + 1) * DH] * SCALE).astype(jnp.bfloat16)
            kh = kfull[:, kvh, :].astype(jnp.bfloat16)
            vh = vfull[:, kvh, :].astype(jnp.bfloat16)
            s = lax.dot_general(qh, kh, (((1,), (1,)), ((), ())),
                                preferred_element_type=jnp.float32)
            mh = jnp.max(s, axis=1, keepdims=True)
            p = jnp.exp(s - mh)
            lh = jnp.sum(p, axis=1, keepdims=True)
            oh = lax.dot_general(p.astype(jnp.bfloat16), vh,
                                 (((1,), (0,)), ((), ())),
                                 preferred_element_type=jnp.float32)
            acc_o[:, h * DH:(h + 1) * DH] = oh
            acc_m[:, h:h + 1] = mh
            acc_l[:, h:h + 1] = lh

        obuf[0] = acc_o[...]
        mlbuf[0, :, 0:HQ] = acc_m[...]
        mlbuf[0, :, HQ:2 * HQ] = acc_l[...]

        for hop in range(N_DEV - 1):
            s_slot = hop % 2
            r_slot = (hop + 1) % 2
            if hop >= 2:
                pl.semaphore_wait(credit_sem, 1)
            rd_o = pltpu.make_async_remote_copy(
                src_ref=obuf.at[s_slot], dst_ref=obuf.at[r_slot],
                send_sem=send_o.at[s_slot], recv_sem=recv_o.at[r_slot],
                device_id=(right,), device_id_type=pl.DeviceIdType.MESH)
            rd_ml = pltpu.make_async_remote_copy(
                src_ref=mlbuf.at[s_slot], dst_ref=mlbuf.at[r_slot],
                send_sem=send_ml.at[s_slot], recv_sem=recv_ml.at[r_slot],
                device_id=(right,), device_id_type=pl.DeviceIdType.MESH)
            rd_o.start()
            rd_ml.start()
            rd_o.wait()
            rd_ml.wait()
            if 1 <= hop <= N_DEV - 3:
                pl.semaphore_signal(credit_sem, inc=1, device_id=(left,),
                                    device_id_type=pl.DeviceIdType.MESH)
            for h in range(HQ):
                m_in = mlbuf[r_slot, :, h:h + 1]
                l_in = mlbuf[r_slot, :, HQ + h:HQ + h + 1]
                o_in = obuf[r_slot, :, h * DH:(h + 1) * DH]
                m_acc = acc_m[:, h:h + 1]
                m_new = jnp.maximum(m_acc, m_in)
                a = jnp.exp(m_acc - m_new)
                b = jnp.exp(m_in - m_new)
                acc_o[:, h * DH:(h + 1) * DH] = (
                    acc_o[:, h * DH:(h + 1) * DH] * a + o_in * b)
                acc_l[:, h:h + 1] = acc_l[:, h:h + 1] * a + l_in * b
                acc_m[:, h:h + 1] = m_new

        for h in range(HQ):
            acc_o[:, h * DH:(h + 1) * DH] = (
                acc_o[:, h * DH:(h + 1) * DH] / acc_l[:, h:h + 1])
        yb = acc_o[...].astype(jnp.bfloat16)
        wo = wo_ref[...].astype(jnp.bfloat16)
        out_ref[0] = jnp.dot(yb, wo, preferred_element_type=jnp.float32)

    return pl.pallas_call(
        body,
        out_shape=jax.ShapeDtypeStruct((1, SQ, DM), jnp.float32),
        in_specs=[pl.BlockSpec(memory_space=pltpu.VMEM)] * 5,
        out_specs=pl.BlockSpec(memory_space=pltpu.VMEM),
        scratch_shapes=[
            pltpu.VMEM((SQ, DM), jnp.float32),
            pltpu.VMEM((SQ, HQ), jnp.float32),
            pltpu.VMEM((SQ, HQ), jnp.float32),
            pltpu.VMEM((2, SQ, DM), jnp.float32),
            pltpu.VMEM((2, SQ, 2 * HQ), jnp.float32),
            pltpu.SemaphoreType.DMA((2,)),
            pltpu.SemaphoreType.DMA((2,)),
            pltpu.SemaphoreType.DMA((2,)),
            pltpu.SemaphoreType.DMA((2,)),
            pltpu.SemaphoreType.REGULAR,
        ],
        compiler_params=pltpu.CompilerParams(collective_id=0),
    )(x, Wq, Wo, K_ext, V_ext)


# baseline (device time: 642242 ns/iter reference)
import jax
import jax.numpy as jnp
from jax import lax
from jax.experimental import pallas as pl
from jax.experimental.pallas import tpu as pltpu

N_DEV = 32
SQ = 256
HQ = 8
HKV = 2
GROUP = HQ // HKV
DH = 128
DM = HQ * DH
SCALE = 0.08838834764831843


def kernel(x, Wq, Wo, K_ext, V_ext):
    skv = K_ext.shape[1]

    def body(x_ref, wq_ref, wo_ref, k_ref, v_ref, out_ref,
             acc_o, acc_m, acc_l, obuf, mlbuf,
             send_o, recv_o, send_ml, recv_ml, credit_sem):
        my = lax.axis_index("i")
        left = lax.rem(my - 1 + N_DEV, N_DEV)
        right = lax.rem(my + 1, N_DEV)

        barrier = pltpu.get_barrier_semaphore()
        pl.semaphore_signal(barrier, inc=1, device_id=(left,),
                            device_id_type=pl.DeviceIdType.MESH)
        pl.semaphore_signal(barrier, inc=1, device_id=(right,),
                            device_id_type=pl.DeviceIdType.MESH)
        pl.semaphore_wait(barrier, 2)

        xv = x_ref[0].astype(jnp.bfloat16)
        wq = wq_ref[...].astype(jnp.bfloat16)
        q = jnp.dot(xv, wq, preferred_element_type=jnp.float32)

        kfull = k_ref[0]
        vfull = v_ref[0]
        for h in range(HQ):
            kvh = h // GROUP
            qh = (q[:, h * DH:(h + 1) * DH] * SCALE).astype(jnp.bfloat16)
            kh = kfull[:, kvh, :].astype(jnp.bfloat16)
            vh = vfull[:, kvh, :].astype(jnp.bfloat16)
            s = lax.dot_general(qh, kh, (((1,), (1,)), ((), ())),
                                preferred_element_type=jnp.float32)
            mh = jnp.max(s, axis=1, keepdims=True)
            p = jnp.exp(s - mh)
            lh = jnp.sum(p, axis=1, keepdims=True)
            oh = lax.dot_general(p.astype(jnp.bfloat16), vh,
                                 (((1,), (0,)), ((), ())),
                                 preferred_element_type=jnp.float32)
            acc_o[:, h * DH:(h + 1) * DH] = oh
            acc_m[:, h:h + 1] = mh
            acc_l[:, h:h + 1] = lh

        obuf[0] = acc_o[...]
        mlbuf[0, :, 0:HQ] = acc_m[...]
        mlbuf[0, :, HQ:2 * HQ] = acc_l[...]

        for hop in range(N_DEV - 1):
            s_slot = hop % 2
            r_slot = (hop + 1) % 2
            if hop >= 2:
                pl.semaphore_wait(credit_sem, 1)
            rd_o = pltpu.make_async_remote_copy(
                src_ref=obuf.at[s_slot], dst_ref=obuf.at[r_slot],
                send_sem=send_o.at[s_slot], recv_sem=recv_o.at[r_slot],
                device_id=(right,), device_id_type=pl.DeviceIdType.MESH)
            rd_ml = pltpu.make_async_remote_copy(
                src_ref=mlbuf.at[s_slot], dst_ref=mlbuf.at[r_slot],
                send_sem=send_ml.at[s_slot], recv_sem=recv_ml.at[r_slot],
                device_id=(right,), device_id_type=pl.DeviceIdType.MESH)
            rd_o.start()
            rd_ml.start()
            rd_o.wait()
            rd_ml.wait()
            if 1 <= hop <= N_DEV - 3:
                pl.semaphore_signal(credit_sem, inc=1, device_id=(left,),
                                    device_id_type=pl.DeviceIdType.MESH)
            for h in range(HQ):
                m_in = mlbuf[r_slot, :, h:h + 1]
                l_in = mlbuf[r_slot, :, HQ + h:HQ + h + 1]
                o_in = obuf[r_slot, :, h * DH:(h + 1) * DH]
                m_acc = acc_m[:, h:h + 1]
                m_new = jnp.maximum(m_acc, m_in)
                a = jnp.exp(m_acc - m_new)
                b = jnp.exp(m_in - m_new)
                acc_o[:, h * DH:(h + 1) * DH] = (
                    acc_o[:, h * DH:(h + 1) * DH] * a + o_in * b)
                acc_l[:, h:h + 1] = acc_l[:, h:h + 1] * a + l_in * b
                acc_m[:, h:h + 1] = m_new

        for h in range(HQ):
            acc_o[:, h * DH:(h + 1) * DH] = (
                acc_o[:, h * DH:(h + 1) * DH] / acc_l[:, h:h + 1])
        yb = acc_o[...].astype(jnp.bfloat16)
        wo = wo_ref[...].astype(jnp.bfloat16)
        out_ref[0] = jnp.dot(yb, wo, preferred_element_type=jnp.float32)

    return pl.pallas_call(
        body,
        out_shape=jax.ShapeDtypeStruct((1, SQ, DM), jnp.float32),
        in_specs=[pl.BlockSpec(memory_space=pltpu.VMEM)] * 5,
        out_specs=pl.BlockSpec(memory_space=pltpu.VMEM),
        scratch_shapes=[
            pltpu.VMEM((SQ, DM), jnp.float32),
            pltpu.VMEM((SQ, HQ), jnp.float32),
            pltpu.VMEM((SQ, HQ), jnp.float32),
            pltpu.VMEM((2, SQ, DM), jnp.float32),
            pltpu.VMEM((2, SQ, 2 * HQ), jnp.float32),
            pltpu.SemaphoreType.DMA((2,)),
            pltpu.SemaphoreType.DMA((2,)),
            pltpu.SemaphoreType.DMA((2,)),
            pltpu.SemaphoreType.DMA((2,)),
            pltpu.SemaphoreType.REGULAR,
        ],
        compiler_params=pltpu.CompilerParams(
            collective_id=0,
            vmem_limit_bytes=100 * 1024 * 1024,
        ),
    )(x, Wq, Wo, K_ext, V_ext)


# device time: 105108 ns/iter; 6.1103x vs baseline; 6.1103x over previous
import jax
import jax.numpy as jnp
from jax import lax
from jax.experimental import pallas as pl
from jax.experimental.pallas import tpu as pltpu

N_DEV = 32
BITS = 5
SQ = 256
HQ = 8
HKV = 2
GROUP = HQ // HKV
DH = 128
DM = HQ * DH
CH = SQ // N_DEV
SCALE = 0.08838834764831843


def kernel(x, Wq, Wo, K_ext, V_ext):
    def body(x_ref, wq_ref, wo_ref, k_ref, v_ref, out_ref,
             acc_o, acc_ml, land_o, land_ml, ystage,
             rs_so, rs_ro, rs_sml, rs_rml, ag_s, ag_r, cp_sem):
        my = lax.axis_index("i")

        barrier = pltpu.get_barrier_semaphore()
        for b in range(BITS):
            partner = my ^ (1 << b)
            pl.semaphore_signal(barrier, inc=1, device_id=(partner,),
                                device_id_type=pl.DeviceIdType.MESH)
        pl.semaphore_wait(barrier, BITS)

        xv = x_ref[0]
        wq = wq_ref[...]
        q = jnp.dot(xv, wq, preferred_element_type=jnp.float32)

        kfull = k_ref[0]
        vfull = v_ref[0]
        for h in range(HQ):
            kvh = h // GROUP
            qh = (q[:, h * DH:(h + 1) * DH] * SCALE).astype(jnp.bfloat16)
            kh = kfull[:, kvh, :]
            vh = vfull[:, kvh, :]
            s = lax.dot_general(qh, kh, (((1,), (1,)), ((), ())),
                                preferred_element_type=jnp.float32)
            mh = jnp.max(s, axis=1, keepdims=True)
            p = jnp.exp(s - mh)
            lh = jnp.sum(p, axis=1, keepdims=True)
            oh = lax.dot_general(p.astype(jnp.bfloat16), vh,
                                 (((1,), (0,)), ((), ())),
                                 preferred_element_type=jnp.float32)
            acc_o[:, h * DH:(h + 1) * DH] = oh
            acc_ml[:, h:h + 1] = mh
            acc_ml[:, HQ + h:HQ + h + 1] = lh

        for k in range(BITS):
            b = BITS - 1 - k
            hrows = (SQ >> k) // 2
            partner = my ^ (1 << b)
            bit = (my >> b) & 1

            def mk_round(koff, soff, k=k, hrows=hrows, partner=partner):
                def _round():
                    rd_o = pltpu.make_async_remote_copy(
                        src_ref=acc_o.at[pl.ds(soff, hrows), :],
                        dst_ref=land_o.at[k, pl.ds(0, hrows), :],
                        send_sem=rs_so.at[k], recv_sem=rs_ro.at[k],
                        device_id=(partner,),
                        device_id_type=pl.DeviceIdType.MESH)
                    rd_ml = pltpu.make_async_remote_copy(
                        src_ref=acc_ml.at[pl.ds(soff, hrows), :],
                        dst_ref=land_ml.at[k, pl.ds(0, hrows), :],
                        send_sem=rs_sml.at[k], recv_sem=rs_rml.at[k],
                        device_id=(partner,),
                        device_id_type=pl.DeviceIdType.MESH)
                    rd_o.start()
                    rd_ml.start()
                    rd_o.wait()
                    rd_ml.wait()
                    for hh in range(HQ):
                        cs = slice(hh * DH, (hh + 1) * DH)
                        m_in = land_ml[k, 0:hrows, hh:hh + 1]
                        l_in = land_ml[k, 0:hrows, HQ + hh:HQ + hh + 1]
                        o_in = land_o[k, 0:hrows, cs]
                        m_acc = acc_ml[koff:koff + hrows, hh:hh + 1]
                        l_acc = acc_ml[koff:koff + hrows,
                                       HQ + hh:HQ + hh + 1]
                        o_acc = acc_o[koff:koff + hrows, cs]
                        m_new = jnp.maximum(m_acc, m_in)
                        ea = jnp.exp(m_acc - m_new)
                        eb = jnp.exp(m_in - m_new)
                        acc_o[0:hrows, cs] = o_acc * ea + o_in * eb
                        acc_ml[0:hrows, HQ + hh:HQ + hh + 1] = (
                            l_acc * ea + l_in * eb)
                        acc_ml[0:hrows, hh:hh + 1] = m_new
                return _round

            pl.when(bit == 0)(mk_round(0, hrows))
            pl.when(bit == 1)(mk_round(hrows, 0))

        for hh in range(HQ):
            cs = slice(hh * DH, (hh + 1) * DH)
            acc_o[0:CH, cs] = (acc_o[0:CH, cs]
                               / acc_ml[0:CH, HQ + hh:HQ + hh + 1])
        yb = acc_o[0:CH, :].astype(jnp.bfloat16)
        wo = wo_ref[...]
        ystage[...] = jnp.dot(yb, wo, preferred_element_type=jnp.float32)
        cp = pltpu.make_async_copy(
            ystage, out_ref.at[0, pl.ds(my * CH, CH), :], cp_sem)
        cp.start()
        cp.wait()

        for k in range(BITS):
            sz = CH << k
            partner = my ^ (1 << k)
            a_start = ((my >> k) << k) * CH
            p_start = ((partner >> k) << k) * CH
            rd = pltpu.make_async_remote_copy(
                src_ref=out_ref.at[0, pl.ds(a_start, sz), :],
                dst_ref=out_ref.at[0, pl.ds(a_start, sz), :],
                send_sem=ag_s.at[k], recv_sem=ag_r.at[k],
                device_id=(partner,), device_id_type=pl.DeviceIdType.MESH)
            rd.start()
            rd.wait_send()
            rcv = pltpu.make_async_remote_copy(
                src_ref=out_ref.at[0, pl.ds(p_start, sz), :],
                dst_ref=out_ref.at[0, pl.ds(p_start, sz), :],
                send_sem=ag_s.at[k], recv_sem=ag_r.at[k],
                device_id=(partner,), device_id_type=pl.DeviceIdType.MESH)
            rcv.wait_recv()

    return pl.pallas_call(
        body,
        out_shape=jax.ShapeDtypeStruct((1, SQ, DM), jnp.float32),
        in_specs=[pl.BlockSpec(memory_space=pltpu.VMEM)] * 5,
        out_specs=pl.BlockSpec(memory_space=pltpu.VMEM),
        scratch_shapes=[
            pltpu.VMEM((SQ, DM), jnp.float32),
            pltpu.VMEM((SQ, 2 * HQ), jnp.float32),
            pltpu.VMEM((BITS, SQ // 2, DM), jnp.float32),
            pltpu.VMEM((BITS, SQ // 2, 2 * HQ), jnp.float32),
            pltpu.VMEM((CH, DM), jnp.float32),
            pltpu.SemaphoreType.DMA((BITS,)),
            pltpu.SemaphoreType.DMA((BITS,)),
            pltpu.SemaphoreType.DMA((BITS,)),
            pltpu.SemaphoreType.DMA((BITS,)),
            pltpu.SemaphoreType.DMA((BITS,)),
            pltpu.SemaphoreType.DMA((BITS,)),
            pltpu.SemaphoreType.DMA,
        ],
        compiler_params=pltpu.CompilerParams(
            collective_id=0,
            vmem_limit_bytes=100 * 1024 * 1024,
        ),
    )(x.astype(jnp.bfloat16), Wq.astype(jnp.bfloat16),
      Wo.astype(jnp.bfloat16), K_ext.astype(jnp.bfloat16),
      V_ext.astype(jnp.bfloat16))


# device time: 95545 ns/iter; 6.7219x vs baseline; 1.1001x over previous
import jax
import jax.numpy as jnp
from jax import lax
from jax.experimental import pallas as pl
from jax.experimental.pallas import tpu as pltpu

N_DEV = 32
BITS = 5
SQ = 256
HQ = 8
HKV = 2
GROUP = HQ // HKV
DH = 128
DM = HQ * DH
CH = SQ // N_DEV
SCALE = 0.08838834764831843


def kernel(x, Wq, Wo, K_ext, V_ext):
    def body(x_ref, wq_ref, wo_ref, k_ref, v_ref, out_ref,
             acc_o, acc_ml, sbuf, land_o, land_f32, land_ml, ystage,
             rs_so, rs_ro, rs_sml, rs_rml, ag_s, ag_r, cp_sem):
        my = lax.axis_index("i")

        barrier = pltpu.get_barrier_semaphore()
        for b in range(BITS):
            partner = my ^ (1 << b)
            pl.semaphore_signal(barrier, inc=1, device_id=(partner,),
                                device_id_type=pl.DeviceIdType.MESH)
        pl.semaphore_wait(barrier, BITS)

        xv = x_ref[0]
        wq = wq_ref[...]
        q = jnp.dot(xv, wq, preferred_element_type=jnp.float32)

        kfull = k_ref[0]
        vfull = v_ref[0]
        for h in range(HQ):
            kvh = h // GROUP
            qh = (q[:, h * DH:(h + 1) * DH] * SCALE).astype(jnp.bfloat16)
            kh = kfull[:, kvh, :]
            vh = vfull[:, kvh, :]
            s = lax.dot_general(qh, kh, (((1,), (1,)), ((), ())),
                                preferred_element_type=jnp.float32)
            mh = jnp.max(s, axis=1, keepdims=True)
            p = jnp.exp(s - mh)
            lh = jnp.sum(p, axis=1, keepdims=True)
            oh = lax.dot_general(p.astype(jnp.bfloat16), vh,
                                 (((1,), (0,)), ((), ())),
                                 preferred_element_type=jnp.float32)
            acc_o[:, h * DH:(h + 1) * DH] = oh
            acc_ml[:, h:h + 1] = mh
            acc_ml[:, HQ + h:HQ + h + 1] = lh

        for k in range(BITS):
            b = BITS - 1 - k
            hrows = (SQ >> k) // 2
            partner = my ^ (1 << b)
            bit = (my >> b) & 1

            use_bf16 = hrows >= 16

            def mk_round(koff, soff, k=k, hrows=hrows, partner=partner,
                         use_bf16=use_bf16):
                def _round():
                    if use_bf16:
                        sbuf[0:hrows, :] = acc_o[
                            soff:soff + hrows, :].astype(jnp.bfloat16)
                        rd_o = pltpu.make_async_remote_copy(
                            src_ref=sbuf.at[pl.ds(0, hrows), :],
                            dst_ref=land_o.at[k, pl.ds(0, hrows), :],
                            send_sem=rs_so.at[k], recv_sem=rs_ro.at[k],
                            device_id=(partner,),
                            device_id_type=pl.DeviceIdType.MESH)
                    else:
                        rd_o = pltpu.make_async_remote_copy(
                            src_ref=acc_o.at[pl.ds(soff, hrows), :],
                            dst_ref=land_f32.at[pl.ds(0, hrows), :],
                            send_sem=rs_so.at[k], recv_sem=rs_ro.at[k],
                            device_id=(partner,),
                            device_id_type=pl.DeviceIdType.MESH)
                    rd_ml = pltpu.make_async_remote_copy(
                        src_ref=acc_ml.at[pl.ds(soff, hrows), :],
                        dst_ref=land_ml.at[k, pl.ds(0, hrows), :],
                        send_sem=rs_sml.at[k], recv_sem=rs_rml.at[k],
                        device_id=(partner,),
                        device_id_type=pl.DeviceIdType.MESH)
                    rd_o.start()
                    rd_ml.start()
                    rd_o.wait()
                    rd_ml.wait()
                    for hh in range(HQ):
                        cs = slice(hh * DH, (hh + 1) * DH)
                        m_in = land_ml[k, 0:hrows, hh:hh + 1]
                        l_in = land_ml[k, 0:hrows, HQ + hh:HQ + hh + 1]
                        if use_bf16:
                            o_in = land_o[k, 0:hrows, cs]
                        else:
                            o_in = land_f32[0:hrows, cs]
                        m_acc = acc_ml[koff:koff + hrows, hh:hh + 1]
                        l_acc = acc_ml[koff:koff + hrows,
                                       HQ + hh:HQ + hh + 1]
                        o_acc = acc_o[koff:koff + hrows, cs]
                        m_new = jnp.maximum(m_acc, m_in)
                        ea = jnp.exp(m_acc - m_new)
                        eb = jnp.exp(m_in - m_new)
                        acc_o[0:hrows, cs] = o_acc * ea + o_in * eb
                        acc_ml[0:hrows, HQ + hh:HQ + hh + 1] = (
                            l_acc * ea + l_in * eb)
                        acc_ml[0:hrows, hh:hh + 1] = m_new
                return _round

            pl.when(bit == 0)(mk_round(0, hrows))
            pl.when(bit == 1)(mk_round(hrows, 0))

        for hh in range(HQ):
            cs = slice(hh * DH, (hh + 1) * DH)
            acc_o[0:CH, cs] = (acc_o[0:CH, cs]
                               / acc_ml[0:CH, HQ + hh:HQ + hh + 1])
        yb = acc_o[0:CH, :].astype(jnp.bfloat16)
        wo = wo_ref[...]
        ystage[...] = jnp.dot(yb, wo, preferred_element_type=jnp.float32)
        cp = pltpu.make_async_copy(
            ystage, out_ref.at[0, pl.ds(my * CH, CH), :], cp_sem)
        cp.start()
        cp.wait()

        for k in range(BITS):
            sz = CH << k
            partner = my ^ (1 << k)
            a_start = ((my >> k) << k) * CH
            p_start = ((partner >> k) << k) * CH
            rd = pltpu.make_async_remote_copy(
                src_ref=out_ref.at[0, pl.ds(a_start, sz), :],
                dst_ref=out_ref.at[0, pl.ds(a_start, sz), :],
                send_sem=ag_s.at[k], recv_sem=ag_r.at[k],
                device_id=(partner,), device_id_type=pl.DeviceIdType.MESH)
            rd.start()
            rd.wait_send()
            rcv = pltpu.make_async_remote_copy(
                src_ref=out_ref.at[0, pl.ds(p_start, sz), :],
                dst_ref=out_ref.at[0, pl.ds(p_start, sz), :],
                send_sem=ag_s.at[k], recv_sem=ag_r.at[k],
                device_id=(partner,), device_id_type=pl.DeviceIdType.MESH)
            rcv.wait_recv()

    return pl.pallas_call(
        body,
        out_shape=jax.ShapeDtypeStruct((1, SQ, DM), jnp.float32),
        in_specs=[pl.BlockSpec(memory_space=pltpu.VMEM)] * 5,
        out_specs=pl.BlockSpec(memory_space=pltpu.VMEM),
        scratch_shapes=[
            pltpu.VMEM((SQ, DM), jnp.float32),
            pltpu.VMEM((SQ, 2 * HQ), jnp.float32),
            pltpu.VMEM((SQ // 2, DM), jnp.bfloat16),
            pltpu.VMEM((BITS, SQ // 2, DM), jnp.bfloat16),
            pltpu.VMEM((CH, DM), jnp.float32),
            pltpu.VMEM((BITS, SQ // 2, 2 * HQ), jnp.float32),
            pltpu.VMEM((CH, DM), jnp.float32),
            pltpu.SemaphoreType.DMA((BITS,)),
            pltpu.SemaphoreType.DMA((BITS,)),
            pltpu.SemaphoreType.DMA((BITS,)),
            pltpu.SemaphoreType.DMA((BITS,)),
            pltpu.SemaphoreType.DMA((BITS,)),
            pltpu.SemaphoreType.DMA((BITS,)),
            pltpu.SemaphoreType.DMA,
        ],
        compiler_params=pltpu.CompilerParams(
            collective_id=0,
            vmem_limit_bytes=100 * 1024 * 1024,
        ),
    )(x.astype(jnp.bfloat16), Wq.astype(jnp.bfloat16),
      Wo.astype(jnp.bfloat16), K_ext.astype(jnp.bfloat16),
      V_ext.astype(jnp.bfloat16))


# device time: 87760 ns/iter; 7.3182x vs baseline; 1.0887x over previous
import jax
import jax.numpy as jnp
from jax import lax
from jax.experimental import pallas as pl
from jax.experimental.pallas import tpu as pltpu

N_DEV = 32
BITS = 5
SQ = 256
HQ = 8
HKV = 2
GROUP = HQ // HKV
DH = 128
DM = HQ * DH
CH = SQ // N_DEV
SCALE = 0.08838834764831843


def kernel(x, Wq, Wo, K_ext, V_ext):
    def body(x_ref, wq_ref, wo_ref, k_ref, v_ref, out_ref,
             acc_o, acc_ml, sbuf, land_o, land_f32, land_ml,
             yst0, ystb, ybuf,
             rs_so, rs_ro, rs_sml, rs_rml, ag_s, ag_r,
             ag0_s, ag0_r, cp_sem):
        my = lax.axis_index("i")

        barrier = pltpu.get_barrier_semaphore()
        for b in range(BITS):
            partner = my ^ (1 << b)
            pl.semaphore_signal(barrier, inc=1, device_id=(partner,),
                                device_id_type=pl.DeviceIdType.MESH)
        pl.semaphore_wait(barrier, BITS)

        xv = x_ref[0]
        wq = wq_ref[...]
        q = jnp.dot(xv, wq, preferred_element_type=jnp.float32)

        kfull = k_ref[0]
        vfull = v_ref[0]
        for h in range(HQ):
            kvh = h // GROUP
            qh = (q[:, h * DH:(h + 1) * DH] * SCALE).astype(jnp.bfloat16)
            kh = kfull[:, kvh, :]
            vh = vfull[:, kvh, :]
            s = lax.dot_general(qh, kh, (((1,), (1,)), ((), ())),
                                preferred_element_type=jnp.float32)
            mh = jnp.max(s, axis=1, keepdims=True)
            p = jnp.exp(s - mh)
            lh = jnp.sum(p, axis=1, keepdims=True)
            oh = lax.dot_general(p.astype(jnp.bfloat16), vh,
                                 (((1,), (0,)), ((), ())),
                                 preferred_element_type=jnp.float32)
            acc_o[:, h * DH:(h + 1) * DH] = oh
            acc_ml[:, h:h + 1] = mh
            acc_ml[:, HQ + h:HQ + h + 1] = lh

        for k in range(BITS):
            b = BITS - 1 - k
            hrows = (SQ >> k) // 2
            partner = my ^ (1 << b)
            bit = (my >> b) & 1

            use_bf16 = hrows >= 16

            def mk_round(koff, soff, k=k, hrows=hrows, partner=partner,
                         use_bf16=use_bf16):
                def _round():
                    if use_bf16:
                        sbuf[0:hrows, :] = acc_o[
                            soff:soff + hrows, :].astype(jnp.bfloat16)
                        rd_o = pltpu.make_async_remote_copy(
                            src_ref=sbuf.at[pl.ds(0, hrows), :],
                            dst_ref=land_o.at[k, pl.ds(0, hrows), :],
                            send_sem=rs_so.at[k], recv_sem=rs_ro.at[k],
                            device_id=(partner,),
                            device_id_type=pl.DeviceIdType.MESH)
                    else:
                        rd_o = pltpu.make_async_remote_copy(
                            src_ref=acc_o.at[pl.ds(soff, hrows), :],
                            dst_ref=land_f32.at[pl.ds(0, hrows), :],
                            send_sem=rs_so.at[k], recv_sem=rs_ro.at[k],
                            device_id=(partner,),
                            device_id_type=pl.DeviceIdType.MESH)
                    rd_ml = pltpu.make_async_remote_copy(
                        src_ref=acc_ml.at[pl.ds(soff, hrows), :],
                        dst_ref=land_ml.at[k, pl.ds(0, hrows), :],
                        send_sem=rs_sml.at[k], recv_sem=rs_rml.at[k],
                        device_id=(partner,),
                        device_id_type=pl.DeviceIdType.MESH)
                    rd_o.start()
                    rd_ml.start()
                    rd_o.wait()
                    rd_ml.wait()
                    for hh in range(HQ):
                        cs = slice(hh * DH, (hh + 1) * DH)
                        m_in = land_ml[k, 0:hrows, hh:hh + 1]
                        l_in = land_ml[k, 0:hrows, HQ + hh:HQ + hh + 1]
                        if use_bf16:
                            o_in = land_o[k, 0:hrows, cs]
                        else:
                            o_in = land_f32[0:hrows, cs]
                        m_acc = acc_ml[koff:koff + hrows, hh:hh + 1]
                        l_acc = acc_ml[koff:koff + hrows,
                                       HQ + hh:HQ + hh + 1]
                        o_acc = acc_o[koff:koff + hrows, cs]
                        m_new = jnp.maximum(m_acc, m_in)
                        ea = jnp.exp(m_acc - m_new)
                        eb = jnp.exp(m_in - m_new)
                        acc_o[0:hrows, cs] = o_acc * ea + o_in * eb
                        acc_ml[0:hrows, HQ + hh:HQ + hh + 1] = (
                            l_acc * ea + l_in * eb)
                        acc_ml[0:hrows, hh:hh + 1] = m_new
                return _round

            pl.when(bit == 0)(mk_round(0, hrows))
            pl.when(bit == 1)(mk_round(hrows, 0))

        for hh in range(HQ):
            cs = slice(hh * DH, (hh + 1) * DH)
            acc_o[0:CH, cs] = (acc_o[0:CH, cs]
                               / acc_ml[0:CH, HQ + hh:HQ + hh + 1])
        yb = acc_o[0:CH, :].astype(jnp.bfloat16)
        wo = wo_ref[...]
        y = jnp.dot(yb, wo, preferred_element_type=jnp.float32)

        p0 = my ^ 1

        def mk_ag0(mine, theirs):
            def _ag0():
                yst0[mine:mine + CH, :] = y
                rd0 = pltpu.make_async_remote_copy(
                    src_ref=yst0.at[pl.ds(mine, CH), :],
                    dst_ref=yst0.at[pl.ds(mine, CH), :],
                    send_sem=ag0_s, recv_sem=ag0_r,
                    device_id=(p0,), device_id_type=pl.DeviceIdType.MESH)
                rd0.start()
                rd0.wait_send()
                rcv0 = pltpu.make_async_remote_copy(
                    src_ref=yst0.at[pl.ds(theirs, CH), :],
                    dst_ref=yst0.at[pl.ds(theirs, CH), :],
                    send_sem=ag0_s, recv_sem=ag0_r,
                    device_id=(p0,), device_id_type=pl.DeviceIdType.MESH)
                rcv0.wait_recv()
            return _ag0

        b0 = my & 1
        pl.when(b0 == 0)(mk_ag0(0, CH))
        pl.when(b0 == 1)(mk_ag0(CH, 0))

        ystb[...] = yst0[...].astype(jnp.bfloat16)
        a1 = ((my >> 1) << 1) * CH
        cp = pltpu.make_async_copy(ystb, ybuf.at[pl.ds(a1, 2 * CH), :],
                                   cp_sem)
        cp.start()
        cp.wait()

        for k in range(1, BITS):
            sz = CH << k
            partner = my ^ (1 << k)
            a_start = ((my >> k) << k) * CH
            p_start = ((partner >> k) << k) * CH
            rd = pltpu.make_async_remote_copy(
                src_ref=ybuf.at[pl.ds(a_start, sz), :],
                dst_ref=ybuf.at[pl.ds(a_start, sz), :],
                send_sem=ag_s.at[k], recv_sem=ag_r.at[k],
                device_id=(partner,), device_id_type=pl.DeviceIdType.MESH)
            rd.start()
            rd.wait_send()
            rcv = pltpu.make_async_remote_copy(
                src_ref=ybuf.at[pl.ds(p_start, sz), :],
                dst_ref=ybuf.at[pl.ds(p_start, sz), :],
                send_sem=ag_s.at[k], recv_sem=ag_r.at[k],
                device_id=(partner,), device_id_type=pl.DeviceIdType.MESH)
            rcv.wait_recv()
        out_ref[0] = ybuf[...].astype(jnp.float32)

    return pl.pallas_call(
        body,
        out_shape=jax.ShapeDtypeStruct((1, SQ, DM), jnp.float32),
        in_specs=[pl.BlockSpec(memory_space=pltpu.VMEM)] * 5,
        out_specs=pl.BlockSpec(memory_space=pltpu.VMEM),
        scratch_shapes=[
            pltpu.VMEM((SQ, DM), jnp.float32),
            pltpu.VMEM((SQ, 2 * HQ), jnp.float32),
            pltpu.VMEM((SQ // 2, DM), jnp.bfloat16),
            pltpu.VMEM((BITS, SQ // 2, DM), jnp.bfloat16),
            pltpu.VMEM((CH, DM), jnp.float32),
            pltpu.VMEM((BITS, SQ // 2, 2 * HQ), jnp.float32),
            pltpu.VMEM((2 * CH, DM), jnp.float32),
            pltpu.VMEM((2 * CH, DM), jnp.bfloat16),
            pltpu.VMEM((SQ, DM), jnp.bfloat16),
            pltpu.SemaphoreType.DMA((BITS,)),
            pltpu.SemaphoreType.DMA((BITS,)),
            pltpu.SemaphoreType.DMA((BITS,)),
            pltpu.SemaphoreType.DMA((BITS,)),
            pltpu.SemaphoreType.DMA((BITS,)),
            pltpu.SemaphoreType.DMA((BITS,)),
            pltpu.SemaphoreType.DMA,
            pltpu.SemaphoreType.DMA,
            pltpu.SemaphoreType.DMA,
        ],
        compiler_params=pltpu.CompilerParams(
            collective_id=0,
            vmem_limit_bytes=100 * 1024 * 1024,
        ),
    )(x.astype(jnp.bfloat16), Wq.astype(jnp.bfloat16),
      Wo.astype(jnp.bfloat16), K_ext.astype(jnp.bfloat16),
      V_ext.astype(jnp.bfloat16))


# device time: 82154 ns/iter; 7.8175x vs baseline; 1.0682x over previous
import jax
import jax.numpy as jnp
from jax import lax
from jax.experimental import pallas as pl
from jax.experimental.pallas import tpu as pltpu

N_DEV = 32
BITS = 5
SQ = 256
HQ = 8
HKV = 2
GROUP = HQ // HKV
DH = 128
DM = HQ * DH
CH = SQ // N_DEV
SCALE = 0.08838834764831843


def kernel(x, Wq, Wo, K_ext, V_ext):
    def body(x_ref, wq_ref, wo_ref, k_ref, v_ref, out_ref,
             acc_o, acc_ml, sbuf, land_o, land_f32, land_ml,
             yst0, ystb, ybuf,
             rs_so, rs_ro, rs_sml, rs_rml, ag_s, ag_r,
             ag0_s, ag0_r, cp_sem):
        my = lax.axis_index("i")

        barrier = pltpu.get_barrier_semaphore()
        for b in range(BITS):
            partner = my ^ (1 << b)
            pl.semaphore_signal(barrier, inc=1, device_id=(partner,),
                                device_id_type=pl.DeviceIdType.MESH)
        pl.semaphore_wait(barrier, BITS)

        xv = x_ref[0]
        wq = wq_ref[...]
        q = jnp.dot(xv, wq, preferred_element_type=jnp.float32)

        kfull = k_ref[0]
        vfull = v_ref[0]
        for h in range(HQ):
            kvh = h // GROUP
            qh = (q[:, h * DH:(h + 1) * DH] * SCALE).astype(jnp.bfloat16)
            kh = kfull[:, kvh, :]
            vh = vfull[:, kvh, :]
            s = lax.dot_general(qh, kh, (((1,), (1,)), ((), ())),
                                preferred_element_type=jnp.float32)
            mh = jnp.max(s, axis=1, keepdims=True)
            p = jnp.exp(s - mh)
            lh = jnp.sum(p, axis=1, keepdims=True)
            oh = lax.dot_general(p.astype(jnp.bfloat16), vh,
                                 (((1,), (0,)), ((), ())),
                                 preferred_element_type=jnp.float32)
            acc_o[:, h * DH:(h + 1) * DH] = oh
            acc_ml[:, h:h + 1] = mh
            acc_ml[:, HQ + h:HQ + h + 1] = lh

        for k in range(BITS):
            b = BITS - 1 - k
            hrows = (SQ >> k) // 2
            partner = my ^ (1 << b)
            bit = (my >> b) & 1

            use_bf16 = hrows >= 16

            def mk_round(koff, soff, k=k, hrows=hrows, partner=partner,
                         use_bf16=use_bf16):
                def _round():
                    if use_bf16:
                        sbuf[0:hrows, :] = acc_o[
                            soff:soff + hrows, :].astype(jnp.bfloat16)
                        rd_o = pltpu.make_async_remote_copy(
                            src_ref=sbuf.at[pl.ds(0, hrows), :],
                            dst_ref=land_o.at[k, pl.ds(0, hrows), :],
                            send_sem=rs_so.at[k], recv_sem=rs_ro.at[k],
                            device_id=(partner,),
                            device_id_type=pl.DeviceIdType.MESH)
                    else:
                        rd_o = pltpu.make_async_remote_copy(
                            src_ref=acc_o.at[pl.ds(soff, hrows), :],
                            dst_ref=land_f32.at[pl.ds(0, hrows), :],
                            send_sem=rs_so.at[k], recv_sem=rs_ro.at[k],
                            device_id=(partner,),
                            device_id_type=pl.DeviceIdType.MESH)
                    rd_ml = pltpu.make_async_remote_copy(
                        src_ref=acc_ml.at[pl.ds(soff, hrows), :],
                        dst_ref=land_ml.at[k, pl.ds(0, hrows), :],
                        send_sem=rs_sml.at[k], recv_sem=rs_rml.at[k],
                        device_id=(partner,),
                        device_id_type=pl.DeviceIdType.MESH)
                    rd_o.start()
                    rd_ml.start()
                    rd_o.wait()
                    rd_ml.wait()
                    for hh in range(HQ):
                        cs = slice(hh * DH, (hh + 1) * DH)
                        m_in = land_ml[k, 0:hrows, hh:hh + 1]
                        l_in = land_ml[k, 0:hrows, HQ + hh:HQ + hh + 1]
                        if use_bf16:
                            o_in = land_o[k, 0:hrows, cs]
                        else:
                            o_in = land_f32[0:hrows, cs]
                        m_acc = acc_ml[koff:koff + hrows, hh:hh + 1]
                        l_acc = acc_ml[koff:koff + hrows,
                                       HQ + hh:HQ + hh + 1]
                        o_acc = acc_o[koff:koff + hrows, cs]
                        m_new = jnp.maximum(m_acc, m_in)
                        ea = jnp.exp(m_acc - m_new)
                        eb = jnp.exp(m_in - m_new)
                        acc_o[0:hrows, cs] = o_acc * ea + o_in * eb
                        acc_ml[0:hrows, HQ + hh:HQ + hh + 1] = (
                            l_acc * ea + l_in * eb)
                        acc_ml[0:hrows, hh:hh + 1] = m_new
                return _round

            pl.when(bit == 0)(mk_round(0, hrows))
            pl.when(bit == 1)(mk_round(hrows, 0))

        for hh in range(HQ):
            cs = slice(hh * DH, (hh + 1) * DH)
            acc_o[0:CH, cs] = (acc_o[0:CH, cs]
                               / acc_ml[0:CH, HQ + hh:HQ + hh + 1])
        yb = acc_o[0:CH, :]
        wo = wo_ref[...]
        y = jnp.dot(yb, wo, preferred_element_type=jnp.float32)

        p0 = my ^ 1

        def mk_ag0(mine, theirs):
            def _ag0():
                yst0[mine:mine + CH, :] = y
                rd0 = pltpu.make_async_remote_copy(
                    src_ref=yst0.at[pl.ds(mine, CH), :],
                    dst_ref=yst0.at[pl.ds(mine, CH), :],
                    send_sem=ag0_s, recv_sem=ag0_r,
                    device_id=(p0,), device_id_type=pl.DeviceIdType.MESH)
                rd0.start()
                rd0.wait_send()
                rcv0 = pltpu.make_async_remote_copy(
                    src_ref=yst0.at[pl.ds(theirs, CH), :],
                    dst_ref=yst0.at[pl.ds(theirs, CH), :],
                    send_sem=ag0_s, recv_sem=ag0_r,
                    device_id=(p0,), device_id_type=pl.DeviceIdType.MESH)
                rcv0.wait_recv()
            return _ag0

        b0 = my & 1
        pl.when(b0 == 0)(mk_ag0(0, CH))
        pl.when(b0 == 1)(mk_ag0(CH, 0))

        ystb[...] = yst0[...].astype(jnp.bfloat16)
        a1 = ((my >> 1) << 1) * CH
        cp = pltpu.make_async_copy(ystb, ybuf.at[pl.ds(a1, 2 * CH), :],
                                   cp_sem)
        cp.start()
        cp.wait()

        for k in range(1, BITS):
            sz = CH << k
            partner = my ^ (1 << k)
            a_start = ((my >> k) << k) * CH
            p_start = ((partner >> k) << k) * CH
            rd = pltpu.make_async_remote_copy(
                src_ref=ybuf.at[pl.ds(a_start, sz), :],
                dst_ref=ybuf.at[pl.ds(a_start, sz), :],
                send_sem=ag_s.at[k], recv_sem=ag_r.at[k],
                device_id=(partner,), device_id_type=pl.DeviceIdType.MESH)
            rd.start()
            rd.wait_send()
            rcv = pltpu.make_async_remote_copy(
                src_ref=ybuf.at[pl.ds(p_start, sz), :],
                dst_ref=ybuf.at[pl.ds(p_start, sz), :],
                send_sem=ag_s.at[k], recv_sem=ag_r.at[k],
                device_id=(partner,), device_id_type=pl.DeviceIdType.MESH)
            rcv.wait_recv()
        out_ref[0] = ybuf[...].astype(jnp.float32)

    return pl.pallas_call(
        body,
        out_shape=jax.ShapeDtypeStruct((1, SQ, DM), jnp.float32),
        in_specs=[pl.BlockSpec(memory_space=pltpu.VMEM)] * 5,
        out_specs=pl.BlockSpec(memory_space=pltpu.VMEM),
        scratch_shapes=[
            pltpu.VMEM((SQ, DM), jnp.float32),
            pltpu.VMEM((SQ, 2 * HQ), jnp.float32),
            pltpu.VMEM((SQ // 2, DM), jnp.bfloat16),
            pltpu.VMEM((BITS, SQ // 2, DM), jnp.bfloat16),
            pltpu.VMEM((CH, DM), jnp.float32),
            pltpu.VMEM((BITS, SQ // 2, 2 * HQ), jnp.float32),
            pltpu.VMEM((2 * CH, DM), jnp.float32),
            pltpu.VMEM((2 * CH, DM), jnp.bfloat16),
            pltpu.VMEM((SQ, DM), jnp.bfloat16),
            pltpu.SemaphoreType.DMA((BITS,)),
            pltpu.SemaphoreType.DMA((BITS,)),
            pltpu.SemaphoreType.DMA((BITS,)),
            pltpu.SemaphoreType.DMA((BITS,)),
            pltpu.SemaphoreType.DMA((BITS,)),
            pltpu.SemaphoreType.DMA((BITS,)),
            pltpu.SemaphoreType.DMA,
            pltpu.SemaphoreType.DMA,
            pltpu.SemaphoreType.DMA,
        ],
        compiler_params=pltpu.CompilerParams(
            collective_id=0,
            vmem_limit_bytes=100 * 1024 * 1024,
        ),
    )(x, Wq, Wo, K_ext.astype(jnp.bfloat16), V_ext.astype(jnp.bfloat16))


# device time: 79954 ns/iter; 8.0326x vs baseline; 1.0275x over previous
import jax
import jax.numpy as jnp
from jax import lax
from jax.experimental import pallas as pl
from jax.experimental.pallas import tpu as pltpu

N_DEV = 32
BITS = 5
SQ = 256
HQ = 8
HKV = 2
GROUP = HQ // HKV
DH = 128
DM = HQ * DH
CH = SQ // N_DEV
SCALE = 0.08838834764831843


def kernel(x, Wq, Wo, K_ext, V_ext):
    def body(x_ref, wq_ref, wo_ref, k_ref, v_ref, out_ref,
             acc_o, acc_ml, sbuf, land_o, land_ml, ystb, ybuf,
             rs_so, rs_ro, rs_sml, rs_rml, ag_s, ag_r, cp_sem):
        my = lax.axis_index("i")

        barrier = pltpu.get_barrier_semaphore()
        for b in range(BITS):
            partner = my ^ (1 << b)
            pl.semaphore_signal(barrier, inc=1, device_id=(partner,),
                                device_id_type=pl.DeviceIdType.MESH)
        pl.semaphore_wait(barrier, BITS)

        xv = x_ref[0]
        wq = wq_ref[...]
        q = jnp.dot(xv, wq, preferred_element_type=jnp.float32)

        kfull = k_ref[0]
        vfull = v_ref[0]
        for h in range(HQ):
            kvh = h // GROUP
            qh = (q[:, h * DH:(h + 1) * DH] * SCALE).astype(jnp.bfloat16)
            kh = kfull[:, kvh, :]
            vh = vfull[:, kvh, :]
            s = lax.dot_general(qh, kh, (((1,), (1,)), ((), ())),
                                preferred_element_type=jnp.float32)
            mh = jnp.max(s, axis=1, keepdims=True)
            p = jnp.exp(s - mh)
            lh = jnp.sum(p, axis=1, keepdims=True)
            oh = lax.dot_general(p.astype(jnp.bfloat16), vh,
                                 (((1,), (0,)), ((), ())),
                                 preferred_element_type=jnp.float32)
            acc_o[:, h * DH:(h + 1) * DH] = oh
            acc_ml[:, h:h + 1] = mh
            acc_ml[:, HQ + h:HQ + h + 1] = lh

        for k in range(BITS - 1):
            b = BITS - 1 - k
            hrows = (SQ >> k) // 2
            partner = my ^ (1 << b)
            bit = (my >> b) & 1

            def mk_round(koff, soff, k=k, hrows=hrows, partner=partner):
                def _round():
                    sbuf[0:hrows, :] = acc_o[
                        soff:soff + hrows, :].astype(jnp.bfloat16)
                    rd_o = pltpu.make_async_remote_copy(
                        src_ref=sbuf.at[pl.ds(0, hrows), :],
                        dst_ref=land_o.at[k, pl.ds(0, hrows), :],
                        send_sem=rs_so.at[k], recv_sem=rs_ro.at[k],
                        device_id=(partner,),
                        device_id_type=pl.DeviceIdType.MESH)
                    rd_ml = pltpu.make_async_remote_copy(
                        src_ref=acc_ml.at[pl.ds(soff, hrows), :],
                        dst_ref=land_ml.at[k, pl.ds(0, hrows), :],
                        send_sem=rs_sml.at[k], recv_sem=rs_rml.at[k],
                        device_id=(partner,),
                        device_id_type=pl.DeviceIdType.MESH)
                    rd_o.start()
                    rd_ml.start()
                    rd_o.wait()
                    rd_ml.wait()
                    for hh in range(HQ):
                        cs = slice(hh * DH, (hh + 1) * DH)
                        m_in = land_ml[k, 0:hrows, hh:hh + 1]
                        l_in = land_ml[k, 0:hrows, HQ + hh:HQ + hh + 1]
                        o_in = land_o[k, 0:hrows, cs]
                        m_acc = acc_ml[koff:koff + hrows, hh:hh + 1]
                        l_acc = acc_ml[koff:koff + hrows,
                                       HQ + hh:HQ + hh + 1]
                        o_acc = acc_o[koff:koff + hrows, cs]
                        m_new = jnp.maximum(m_acc, m_in)
                        ea = jnp.exp(m_acc - m_new)
                        eb = jnp.exp(m_in - m_new)
                        acc_o[0:hrows, cs] = o_acc * ea + o_in * eb
                        acc_ml[0:hrows, HQ + hh:HQ + hh + 1] = (
                            l_acc * ea + l_in * eb)
                        acc_ml[0:hrows, hh:hh + 1] = m_new
                return _round

            pl.when(bit == 0)(mk_round(0, hrows))
            pl.when(bit == 1)(mk_round(hrows, 0))

        p0 = my ^ 1
        sbuf[0:2 * CH, :] = acc_o[0:2 * CH, :].astype(jnp.bfloat16)
        rd4_o = pltpu.make_async_remote_copy(
            src_ref=sbuf.at[pl.ds(0, 2 * CH), :],
            dst_ref=land_o.at[BITS - 1, pl.ds(0, 2 * CH), :],
            send_sem=rs_so.at[BITS - 1], recv_sem=rs_ro.at[BITS - 1],
            device_id=(p0,), device_id_type=pl.DeviceIdType.MESH)
        rd4_ml = pltpu.make_async_remote_copy(
            src_ref=acc_ml.at[pl.ds(0, 2 * CH), :],
            dst_ref=land_ml.at[BITS - 1, pl.ds(0, 2 * CH), :],
            send_sem=rs_sml.at[BITS - 1], recv_sem=rs_rml.at[BITS - 1],
            device_id=(p0,), device_id_type=pl.DeviceIdType.MESH)
        rd4_o.start()
        rd4_ml.start()
        rd4_o.wait()
        rd4_ml.wait()
        for hh in range(HQ):
            cs = slice(hh * DH, (hh + 1) * DH)
            m_in = land_ml[BITS - 1, 0:2 * CH, hh:hh + 1]
            l_in = land_ml[BITS - 1, 0:2 * CH, HQ + hh:HQ + hh + 1]
            o_in = land_o[BITS - 1, 0:2 * CH, cs]
            m_acc = acc_ml[0:2 * CH, hh:hh + 1]
            l_acc = acc_ml[0:2 * CH, HQ + hh:HQ + hh + 1]
            o_acc = acc_o[0:2 * CH, cs]
            m_new = jnp.maximum(m_acc, m_in)
            ea = jnp.exp(m_acc - m_new)
            eb = jnp.exp(m_in - m_new)
            l_new = l_acc * ea + l_in * eb
            acc_o[0:2 * CH, cs] = (o_acc * ea + o_in * eb) / l_new

        yb = acc_o[0:2 * CH, :]
        wo = wo_ref[...]
        y = jnp.dot(yb, wo, preferred_element_type=jnp.float32)
        ystb[...] = y.astype(jnp.bfloat16)
        a1 = ((my >> 1) << 1) * CH
        cp = pltpu.make_async_copy(ystb, ybuf.at[pl.ds(a1, 2 * CH), :],
                                   cp_sem)
        cp.start()
        cp.wait()

        for k in range(1, BITS):
            sz = CH << k
            partner = my ^ (1 << k)
            a_start = ((my >> k) << k) * CH
            p_start = ((partner >> k) << k) * CH
            rd = pltpu.make_async_remote_copy(
                src_ref=ybuf.at[pl.ds(a_start, sz), :],
                dst_ref=ybuf.at[pl.ds(a_start, sz), :],
                send_sem=ag_s.at[k], recv_sem=ag_r.at[k],
                device_id=(partner,), device_id_type=pl.DeviceIdType.MESH)
            rd.start()
            rd.wait_send()
            rcv = pltpu.make_async_remote_copy(
                src_ref=ybuf.at[pl.ds(p_start, sz), :],
                dst_ref=ybuf.at[pl.ds(p_start, sz), :],
                send_sem=ag_s.at[k], recv_sem=ag_r.at[k],
                device_id=(partner,), device_id_type=pl.DeviceIdType.MESH)
            rcv.wait_recv()
        out_ref[0] = ybuf[...].astype(jnp.float32)

    return pl.pallas_call(
        body,
        out_shape=jax.ShapeDtypeStruct((1, SQ, DM), jnp.float32),
        in_specs=[pl.BlockSpec(memory_space=pltpu.VMEM)] * 5,
        out_specs=pl.BlockSpec(memory_space=pltpu.VMEM),
        scratch_shapes=[
            pltpu.VMEM((SQ, DM), jnp.float32),
            pltpu.VMEM((SQ, 2 * HQ), jnp.float32),
            pltpu.VMEM((SQ // 2, DM), jnp.bfloat16),
            pltpu.VMEM((BITS, SQ // 2, DM), jnp.bfloat16),
            pltpu.VMEM((BITS, SQ // 2, 2 * HQ), jnp.float32),
            pltpu.VMEM((2 * CH, DM), jnp.bfloat16),
            pltpu.VMEM((SQ, DM), jnp.bfloat16),
            pltpu.SemaphoreType.DMA((BITS,)),
            pltpu.SemaphoreType.DMA((BITS,)),
            pltpu.SemaphoreType.DMA((BITS,)),
            pltpu.SemaphoreType.DMA((BITS,)),
            pltpu.SemaphoreType.DMA((BITS,)),
            pltpu.SemaphoreType.DMA((BITS,)),
            pltpu.SemaphoreType.DMA,
        ],
        compiler_params=pltpu.CompilerParams(
            collective_id=0,
            vmem_limit_bytes=100 * 1024 * 1024,
        ),
    )(x, Wq, Wo, K_ext.astype(jnp.bfloat16), V_ext.astype(jnp.bfloat16))
